# baseline (speedup 1.0000x reference)
import functools

import jax
import jax.numpy as jnp
from jax import lax
from jax.experimental import pallas as pl
from jax.experimental.pallas import tpu as pltpu
from jax.experimental.pallas import tpu_sc as plsc

EMB_SZ = 64
NW = 32
CHUNK = 128


def _emb_kernel(n_chunks, b_per_w):
    mesh = plsc.VectorSubcoreMesh(core_axis_name="c", subcore_axis_name="s")

    @functools.partial(
        pl.kernel,
        out_type=jax.ShapeDtypeStruct((NW * b_per_w, EMB_SZ), jnp.float32),
        mesh=mesh,
        scratch_types=[
            pltpu.VMEM((n_chunks, CHUNK), jnp.int32),
            pltpu.VMEM((CHUNK, EMB_SZ), jnp.float32),
            pltpu.SemaphoreType.DMA,
        ],
        compiler_params=pltpu.CompilerParams(use_tc_tiling_on_sc=False),
    )
    def emb(idx_hbm, table_hbm, out_hbm, idx_v, rows_v, sem):
        wid = lax.axis_index("s") * 2 + lax.axis_index("c")
        pltpu.sync_copy(idx_hbm.at[wid], idx_v)
        base = wid * b_per_w

        def body(j):
            pltpu.async_copy(table_hbm.at[idx_v.at[j]], rows_v, sem).wait()
            pltpu.sync_copy(rows_v, out_hbm.at[pl.ds(base + j * CHUNK, CHUNK)])

        pl.loop(0, n_chunks)(body)

    return emb


def kernel(input_tensor, W):
    n_rows, n_cols = input_tensor.shape
    B = n_rows * n_cols
    b_per_w = B // NW
    n_chunks = b_per_w // CHUNK
    idx = input_tensor.reshape(NW, n_chunks, CHUNK).astype(jnp.int32)
    out = _emb_kernel(n_chunks, b_per_w)(idx, W)
    return out.reshape(n_rows, n_cols, EMB_SZ)
